# baseline (speedup 1.0000x reference)
import functools

import jax
import jax.numpy as jnp
from jax.experimental import pallas as pl
from jax.experimental.pallas import tpu as pltpu

_CODE_NUM = 8
_D = 3
_IDX = 1


def _interp_kernel(coords_ref, code_ref, out_ref, *, cn, c, tp):
    scaled = (coords_ref[...] + 1.0) * ((cn - 1) / 2.0)
    grid_i = jax.lax.broadcasted_iota(jnp.int32, (cn, tp), 0).astype(jnp.float32)

    def hat(j):
        return jnp.maximum(0.0, 1.0 - jnp.abs(grid_i - scaled[j:j + 1, :]))

    h0 = hat(0)
    h1 = hat(1)
    h2 = hat(2)

    w_low = (h1[:, None, :] * h0[None, :, :]).reshape(cn * cn, tp)

    a = jnp.dot(code_ref[...], w_low,
                preferred_element_type=jnp.float32)

    acc = a[0:c, :] * h2[0:1, :]
    for i in range(1, cn):
        acc = acc + a[i * c:(i + 1) * c, :] * h2[i:i + 1, :]

    out_ref[...] = acc.T


@jax.jit
def kernel(coords, shape_code):
    b, p, d = coords.shape
    c = shape_code.shape[0]
    cn = _CODE_NUM
    nblk = cn ** d

    npts = b * p
    tp = 1024
    npts_pad = pl.cdiv(npts, tp) * tp

    coords_t = jnp.pad(coords.reshape(npts, d).T.astype(jnp.float32),
                       ((0, 8 - d), (0, npts_pad - npts)))

    code = jax.lax.slice_in_dim(shape_code, _IDX * nblk, (_IDX + 1) * nblk, axis=1)
    code_r = (code.astype(jnp.float32)
              .reshape(c, cn, cn * cn)
              .transpose(1, 0, 2)
              .reshape(cn * c, cn * cn))

    kernel_fn = functools.partial(_interp_kernel, cn=cn, c=c, tp=tp)

    out = pl.pallas_call(
        kernel_fn,
        out_shape=jax.ShapeDtypeStruct((npts_pad, c), jnp.float32),
        grid=(npts_pad // tp,),
        in_specs=[
            pl.BlockSpec((8, tp), lambda i: (0, i)),
            pl.BlockSpec((cn * c, cn * cn), lambda i: (0, 0)),
        ],
        out_specs=pl.BlockSpec((tp, c), lambda i: (i, 0)),
        compiler_params=pltpu.CompilerParams(
            dimension_semantics=("parallel",),
            vmem_limit_bytes=64 * 1024 * 1024,
        ),
    )(coords_t, code_r)

    return out[:npts].reshape(b, p, c)


# speedup vs baseline: 1.1889x; 1.1889x over previous
import functools

import jax
import jax.numpy as jnp
from jax.experimental import pallas as pl
from jax.experimental.pallas import tpu as pltpu

_CODE_NUM = 8
_D = 3
_IDX = 1


def _interp_kernel(coords_ref, code_ref, out_ref, *, cn, c, tp):
    scaled = (coords_ref[...] + 1.0) * ((cn - 1) / 2.0)
    grid_i = jax.lax.broadcasted_iota(jnp.int32, (cn, tp), 0).astype(jnp.float32)

    def hat(j):
        return jnp.maximum(0.0, 1.0 - jnp.abs(grid_i - scaled[j:j + 1, :]))

    h0 = hat(0)
    h1 = hat(1)
    h2 = hat(2)

    w_low = (h1[:, None, :] * h0[None, :, :]).reshape(cn * cn, tp)

    a = jnp.dot(code_ref[...], w_low,
                preferred_element_type=jnp.float32)

    acc = a[0:c, :] * h2[0:1, :]
    for i in range(1, cn):
        acc = acc + a[i * c:(i + 1) * c, :] * h2[i:i + 1, :]

    out_ref[...] = acc


@jax.jit
def kernel(coords, shape_code):
    b, p, d = coords.shape
    c = shape_code.shape[0]
    cn = _CODE_NUM
    nblk = cn ** d

    npts = b * p
    tp = 1024
    npts_pad = pl.cdiv(npts, tp) * tp

    coords_t = jnp.pad(coords.reshape(npts, d).T.astype(jnp.float32),
                       ((0, 8 - d), (0, npts_pad - npts)))

    code = jax.lax.slice_in_dim(shape_code, _IDX * nblk, (_IDX + 1) * nblk, axis=1)
    code_r = (code.astype(jnp.float32)
              .reshape(c, cn, cn * cn)
              .transpose(1, 0, 2)
              .reshape(cn * c, cn * cn))

    kernel_fn = functools.partial(_interp_kernel, cn=cn, c=c, tp=tp)

    out = pl.pallas_call(
        kernel_fn,
        out_shape=jax.ShapeDtypeStruct((c, npts_pad), jnp.float32),
        grid=(npts_pad // tp,),
        in_specs=[
            pl.BlockSpec((8, tp), lambda i: (0, i)),
            pl.BlockSpec((cn * c, cn * cn), lambda i: (0, 0)),
        ],
        out_specs=pl.BlockSpec((c, tp), lambda i: (0, i)),
        compiler_params=pltpu.CompilerParams(
            dimension_semantics=("parallel",),
            vmem_limit_bytes=64 * 1024 * 1024,
        ),
    )(coords_t, code_r)

    return out[:, :npts].T.reshape(b, p, c)


# speedup vs baseline: 2.5044x; 2.1064x over previous
import functools

import jax
import jax.numpy as jnp
from jax.experimental import pallas as pl
from jax.experimental.pallas import tpu as pltpu

_CODE_NUM = 8
_D = 3
_IDX = 1


def _interp_kernel(coords_ref, code_ref, out_ref, *, cn, c, tp):
    scaled = (coords_ref[...] + 1.0) * ((cn - 1) / 2.0)
    grid_i = jax.lax.broadcasted_iota(jnp.int32, (cn, tp), 0).astype(jnp.float32)

    def hat(j):
        return jnp.maximum(0.0, 1.0 - jnp.abs(grid_i - scaled[j:j + 1, :]))

    h0 = hat(0)
    h1 = hat(1)
    h2 = hat(2)

    w_low = (h1[:, None, :] * h0[None, :, :]).reshape(cn * cn, tp)

    a = jnp.dot(code_ref[...], w_low,
                preferred_element_type=jnp.float32)

    acc = a[0:c, :] * h2[0:1, :]
    for i in range(1, cn):
        acc = acc + a[i * c:(i + 1) * c, :] * h2[i:i + 1, :]

    out_ref[...] = acc


@jax.jit
def kernel(coords, shape_code):
    b, p, d = coords.shape
    c = shape_code.shape[0]
    cn = _CODE_NUM
    nblk = cn ** d

    npts = b * p
    tp = 4096
    npts_pad = pl.cdiv(npts, tp) * tp

    coords_t = jnp.pad(coords.reshape(npts, d).T.astype(jnp.float32),
                       ((0, 8 - d), (0, npts_pad - npts)))

    code = jax.lax.slice_in_dim(shape_code, _IDX * nblk, (_IDX + 1) * nblk, axis=1)
    code_r = (code.astype(jnp.float32)
              .reshape(c, cn, cn * cn)
              .transpose(1, 0, 2)
              .reshape(cn * c, cn * cn))

    kernel_fn = functools.partial(_interp_kernel, cn=cn, c=c, tp=tp)

    out = pl.pallas_call(
        kernel_fn,
        out_shape=jax.ShapeDtypeStruct((c, npts_pad), jnp.float32),
        grid=(npts_pad // tp,),
        in_specs=[
            pl.BlockSpec((8, tp), lambda i: (0, i)),
            pl.BlockSpec((cn * c, cn * cn), lambda i: (0, 0)),
        ],
        out_specs=pl.BlockSpec((c, tp), lambda i: (0, i)),
        compiler_params=pltpu.CompilerParams(
            dimension_semantics=("parallel",),
            vmem_limit_bytes=64 * 1024 * 1024,
        ),
    )(coords_t, code_r)

    return out[:, :npts].T.reshape(b, p, c)


# speedup vs baseline: 2.9725x; 1.1869x over previous
import functools

import jax
import jax.numpy as jnp
from jax.experimental import pallas as pl
from jax.experimental.pallas import tpu as pltpu

_CODE_NUM = 8
_D = 3
_IDX = 1


def _interp_kernel(coords_ref, code_ref, out_ref, *, cn, c, tp):
    scaled = (coords_ref[...] + 1.0) * ((cn - 1) / 2.0)
    grid_i = jax.lax.broadcasted_iota(jnp.int32, (cn, tp), 0).astype(jnp.float32)

    def hat(j):
        return jnp.maximum(0.0, 1.0 - jnp.abs(grid_i - scaled[j:j + 1, :]))

    h0 = hat(0)
    h1 = hat(1)
    h2 = hat(2)

    w_low = (h1[:, None, :] * h0[None, :, :]).reshape(cn * cn, tp)

    a = jnp.dot(code_ref[...], w_low,
                preferred_element_type=jnp.float32)

    acc = a[0:c, :] * h2[0:1, :]
    for i in range(1, cn):
        acc = acc + a[i * c:(i + 1) * c, :] * h2[i:i + 1, :]

    out_ref[...] = acc


@jax.jit
def kernel(coords, shape_code):
    b, p, d = coords.shape
    c = shape_code.shape[0]
    cn = _CODE_NUM
    nblk = cn ** d

    npts = b * p
    tp = 8192
    npts_pad = pl.cdiv(npts, tp) * tp

    coords_t = jnp.pad(coords.reshape(npts, d).T.astype(jnp.float32),
                       ((0, 8 - d), (0, npts_pad - npts)))

    code = jax.lax.slice_in_dim(shape_code, _IDX * nblk, (_IDX + 1) * nblk, axis=1)
    code_r = (code.astype(jnp.float32)
              .reshape(c, cn, cn * cn)
              .transpose(1, 0, 2)
              .reshape(cn * c, cn * cn))

    kernel_fn = functools.partial(_interp_kernel, cn=cn, c=c, tp=tp)

    out = pl.pallas_call(
        kernel_fn,
        out_shape=jax.ShapeDtypeStruct((c, npts_pad), jnp.float32),
        grid=(npts_pad // tp,),
        in_specs=[
            pl.BlockSpec((8, tp), lambda i: (0, i)),
            pl.BlockSpec((cn * c, cn * cn), lambda i: (0, 0)),
        ],
        out_specs=pl.BlockSpec((c, tp), lambda i: (0, i)),
        compiler_params=pltpu.CompilerParams(
            dimension_semantics=("parallel",),
            vmem_limit_bytes=64 * 1024 * 1024,
        ),
    )(coords_t, code_r)

    return out[:, :npts].T.reshape(b, p, c)
